# baseline (speedup 1.0000x reference)
import jax
import jax.numpy as jnp
from jax import lax
from jax.experimental import pallas as pl
from jax.experimental.pallas import tpu as pltpu
from jax.experimental.pallas import tpu_sc as plsc

N_NODES = 10000
N_EDGES = 320000
D = 128
MEM = 256

NC = 1
NS = 16
NW = NC * NS
NP = 10240
RPT = NP // NS
EPT = N_EDGES // NW

CH = 40
NCHUNK = EPT // CH
R = 4
A = 2
F = 5
K2 = 8
NGRP = (NCHUNK + K2 - 1) // K2

DCH = 80
DNCHUNK = EPT // DCH
DR = 8

_HIGH = lax.Precision.HIGHEST


def _mesh():
    return plsc.VectorSubcoreMesh(
        core_axis_name="c", subcore_axis_name="s",
        num_cores=NC, num_subcores=NS)


def _deg_body(ei_hbm, out_hbm, dstf, ones_v, idxr, zb, deg_sh, *sems):
    isem = sems[0]
    cid = lax.axis_index("c")
    sid = lax.axis_index("s")
    wid = cid * NS + sid
    base = wid * EPT

    def _fill(i, _):
        zb[pl.ds(i * 16, 16)] = jnp.zeros((16,), jnp.float32)
        return 0
    lax.fori_loop(0, RPT // 16, _fill, 0)
    for k in range(DCH // 16):
        ones_v[pl.ds(k * 16, 16)] = jnp.ones((16,), jnp.float32)

    pltpu.sync_copy(zb, deg_sh.at[pl.ds(sid * RPT, RPT)])
    plsc.subcore_barrier()

    pltpu.async_copy(ei_hbm.at[pl.ds(N_EDGES + base, EPT)], dstf, isem).wait()

    def _chunk(i, _):
        for k in range(DCH // 16):
            idxr[0, pl.ds(k * 16, 16)] = dstf[pl.ds(i * DCH + k * 16, 16)]
        pltpu.sync_copy(ones_v, deg_sh.at[idxr.at[0]], add=True)
        return 0
    lax.fori_loop(0, DNCHUNK, _chunk, 0)

    plsc.subcore_barrier()
    pltpu.sync_copy(deg_sh.at[pl.ds(sid * RPT, RPT)],
                    out_hbm.at[cid, pl.ds(sid * RPT, RPT)])


def _deg_call(edge_index):
    f = pl.kernel(
        _deg_body,
        out_type=jax.ShapeDtypeStruct((NC, NP), jnp.float32),
        mesh=_mesh(),
        scratch_types=[
            pltpu.VMEM((EPT,), jnp.int32),
            pltpu.VMEM((DCH,), jnp.float32),
            pltpu.VMEM((DR, DCH), jnp.int32),
            pltpu.VMEM((RPT,), jnp.float32),
            pltpu.VMEM_SHARED((NP,), jnp.float32),
        ] + [pltpu.SemaphoreType.DMA],
    )
    return f(edge_index)


def _edge_body(ei_hbm, yw_hbm, out_hbm, sidx, didx, rows, *sems):
    isems = sems[:K2]
    gsems = sems[K2:K2 + R]
    ssems = sems[K2 + R:K2 + 2 * R]
    acc_sh = sems[K2 + 2 * R]
    cid = lax.axis_index("c")
    sid = lax.axis_index("s")
    wid = cid * NS + sid
    base = wid * EPT

    def _fire_idx(slot, i):
        pltpu.async_copy(ei_hbm.at[pl.ds(base + i * CH, CH)],
                         sidx.at[slot], isems[slot])
        pltpu.async_copy(ei_hbm.at[pl.ds(N_EDGES + base + i * CH, CH)],
                         didx.at[slot], isems[slot])

    def _wait_idx(slot, i):
        pltpu.make_async_copy(ei_hbm.at[pl.ds(base + i * CH, CH)],
                              sidx.at[slot], isems[slot]).wait()
        pltpu.make_async_copy(ei_hbm.at[pl.ds(N_EDGES + base + i * CH, CH)],
                              didx.at[slot], isems[slot]).wait()

    def _fire_gather(slot, rslot):
        pltpu.async_copy(yw_hbm.at[sidx.at[slot]], rows.at[rslot],
                         gsems[rslot])

    def _wait_gather(slot, rslot):
        pltpu.make_async_copy(yw_hbm.at[sidx.at[slot]], rows.at[rslot],
                              gsems[rslot]).wait()

    def _fire_scat(slot, rslot):
        pltpu.async_copy(rows.at[rslot], acc_sh.at[didx.at[slot]],
                         ssems[rslot], add=True)

    def _wait_scat(slot, rslot):
        pltpu.make_async_copy(rows.at[rslot], acc_sh.at[didx.at[slot]],
                              ssems[rslot]).wait()

    def _fill(i, _):
        r = i // (D // 16)
        c = i % (D // 16)
        rows[0, r, pl.ds(c * 16, 16)] = jnp.zeros((16,), jnp.float32)
        return 0
    lax.fori_loop(0, CH * (D // 16), _fill, 0)
    for k in range(RPT // CH):
        pltpu.sync_copy(rows.at[0], acc_sh.at[pl.ds(sid * RPT + k * CH, CH), :])
    plsc.subcore_barrier()

    for j in range(F):
        _fire_idx(j, j)
    for j in range(A):
        _wait_idx(j, j)
        _fire_gather(j, j)

    def _group(g, _):
        for b in range(K2):
            i = g * K2 + b

            @pl.when(i < NCHUNK)
            def _():
                _wait_gather(b, b % R)
                _fire_scat(b, b % R)

            @pl.when((i >= A) & (i < NCHUNK + A))
            def _():
                _wait_scat((b - A) % K2, (b - A) % R)

            @pl.when(i + A < NCHUNK)
            def _():
                _wait_idx((b + A) % K2, i + A)
                _fire_gather((b + A) % K2, (b + A) % R)

            @pl.when(i + F < NCHUNK)
            def _():
                _fire_idx((b + F) % K2, i + F)
        return 0
    lax.fori_loop(0, NGRP + 1, _group, 0)

    plsc.subcore_barrier()
    pltpu.sync_copy(acc_sh.at[pl.ds(sid * RPT, RPT), :],
                    out_hbm.at[cid, pl.ds(sid * RPT, RPT), :])


def _edge_call(edge_index, yw):
    f = pl.kernel(
        _edge_body,
        out_type=jax.ShapeDtypeStruct((NC, NP, D), jnp.float32),
        mesh=_mesh(),
        scratch_types=(
            [pltpu.VMEM((K2, CH), jnp.int32),
             pltpu.VMEM((K2, CH), jnp.int32),
             pltpu.VMEM((R, CH, D), jnp.float32)]
            + [pltpu.SemaphoreType.DMA] * (K2 + 2 * R)
            + [pltpu.VMEM_SHARED((NP, D), jnp.float32)]
        ),
    )
    return f(edge_index, yw)


def _wgen_body(mw_ref, wih_ref, bih_ref, bhh_ref, wwt_ref, bwt_ref, nw_ref):
    h = MEM
    dn = (((1,), (1,)), ((), ()))
    gi = lax.dot_general(mw_ref[...], wih_ref[...], dn,
                         precision=_HIGH) + bih_ref[...]
    gh = bhh_ref[...]
    r = jax.nn.sigmoid(gi[:, 0:h] + gh[:, 0:h])
    z = jax.nn.sigmoid(gi[:, h:2 * h] + gh[:, h:2 * h])
    n = jnp.tanh(gi[:, 2 * h:] + r * gh[:, 2 * h:])
    um = (1.0 - z) * n
    nw_ref[...] = lax.dot_general(um, wwt_ref[...], dn,
                                  precision=_HIGH) + bwt_ref[...]


def _wgen_call(memory_weights, w_ih, b_ih, b_hh, W_wt, b_wt):
    f = pl.pallas_call(
        _wgen_body,
        out_shape=jax.ShapeDtypeStruct((1, D * D), jnp.float32),
    )
    return f(memory_weights.reshape(1, MEM), w_ih,
             b_ih.reshape(1, 3 * MEM), b_hh.reshape(1, 3 * MEM),
             W_wt, b_wt.reshape(1, D * D))


_NB = 1000


def _deg_sum(deg_ref):
    d = deg_ref[0]
    for c in range(1, NC):
        d = d + deg_ref[c]
    return d


def _yw_body(x_ref, w_ref, deg_ref, yw_ref):
    dis = lax.rsqrt(_deg_sum(deg_ref) + 1.0)
    xw = lax.dot_general(x_ref[...], w_ref[...],
                         (((1,), (1,)), ((), ())), precision=_HIGH)
    yw_ref[...] = xw * dis


def _yw_call(x, W2, deg3):
    grid = N_NODES // _NB
    f = pl.pallas_call(
        _yw_body,
        grid=(grid,),
        in_specs=[
            pl.BlockSpec((_NB, D), lambda j: (j, 0)),
            pl.BlockSpec((D, D), lambda j: (0, 0)),
            pl.BlockSpec((NC, _NB, 1), lambda j: (0, j, 0)),
        ],
        out_specs=pl.BlockSpec((_NB, D), lambda j: (j, 0)),
        out_shape=jax.ShapeDtypeStruct((N_NODES, D), jnp.float32),
    )
    return f(x, W2, deg3)


def _comb_body(parts_ref, yw_ref, deg_ref, bias_ref, out_ref):
    dis = lax.rsqrt(_deg_sum(deg_ref) + 1.0)
    s = parts_ref[0] + yw_ref[...]
    for c in range(1, NC):
        s = s + parts_ref[c]
    out_ref[...] = dis * s + bias_ref[...]


def _comb_call(parts, yw, deg3, gcn_bias):
    grid = N_NODES // _NB
    f = pl.pallas_call(
        _comb_body,
        grid=(grid,),
        in_specs=[
            pl.BlockSpec((NC, _NB, D), lambda j: (0, j, 0)),
            pl.BlockSpec((_NB, D), lambda j: (j, 0)),
            pl.BlockSpec((NC, _NB, 1), lambda j: (0, j, 0)),
            pl.BlockSpec((1, D), lambda j: (0, 0)),
        ],
        out_specs=pl.BlockSpec((_NB, D), lambda j: (j, 0)),
        out_shape=jax.ShapeDtypeStruct((N_NODES, D), jnp.float32),
    )
    return f(parts, yw, deg3, gcn_bias.reshape(1, D))


def kernel(x, edge_index, memory_weights, w_ih, w_hh, b_ih, b_hh,
           W_wt, b_wt, gcn_bias):
    ei_flat = edge_index.reshape(2 * N_EDGES)
    deg = _deg_call(ei_flat)
    deg3 = deg.reshape(NC, NP, 1)
    nw = _wgen_call(memory_weights, w_ih, b_ih, b_hh, W_wt, b_wt)
    W2 = nw.reshape(D, D)
    yw = _yw_call(x, W2, deg3)
    parts = _edge_call(ei_flat, yw)
    return _comb_call(parts, yw, deg3, gcn_bias)


# speedup vs baseline: 1.2804x; 1.2804x over previous
import jax
import jax.numpy as jnp
from jax import lax
from jax.experimental import pallas as pl
from jax.experimental.pallas import tpu as pltpu
from jax.experimental.pallas import tpu_sc as plsc

N_NODES = 10000
N_EDGES = 320000
D = 128
MEM = 256

NC = 1
NS = 16
NW = NC * NS
NP = 10240
RPT = NP // NS
EPT = N_EDGES // NW

CH = 80
NCHUNK = EPT // CH
R = 4
A = 3
W = R - A
F = 6
K2 = 8
NGRP = (NCHUNK + K2 - 1) // K2

DCH = 80
DNCHUNK = EPT // DCH
DR = 8

_HIGH = lax.Precision.HIGHEST


def _mesh():
    return plsc.VectorSubcoreMesh(
        core_axis_name="c", subcore_axis_name="s",
        num_cores=NC, num_subcores=NS)


def _deg_body(ei_hbm, out_hbm, dstf, ones_v, idxr, zb, deg_sh, *sems):
    isem = sems[0]
    cid = lax.axis_index("c")
    sid = lax.axis_index("s")
    wid = cid * NS + sid
    base = wid * EPT

    def _fill(i, _):
        zb[pl.ds(i * 16, 16)] = jnp.zeros((16,), jnp.float32)
        return 0
    lax.fori_loop(0, RPT // 16, _fill, 0)
    for k in range(DCH // 16):
        ones_v[pl.ds(k * 16, 16)] = jnp.ones((16,), jnp.float32)

    pltpu.sync_copy(zb, deg_sh.at[pl.ds(sid * RPT, RPT)])
    plsc.subcore_barrier()

    pltpu.async_copy(ei_hbm.at[pl.ds(N_EDGES + base, EPT)], dstf, isem).wait()

    def _chunk(i, _):
        for k in range(DCH // 16):
            idxr[0, pl.ds(k * 16, 16)] = dstf[pl.ds(i * DCH + k * 16, 16)]
        pltpu.sync_copy(ones_v, deg_sh.at[idxr.at[0]], add=True)
        return 0
    lax.fori_loop(0, DNCHUNK, _chunk, 0)

    plsc.subcore_barrier()
    pltpu.sync_copy(deg_sh.at[pl.ds(sid * RPT, RPT)],
                    out_hbm.at[cid, pl.ds(sid * RPT, RPT)])


def _deg_call(edge_index):
    f = pl.kernel(
        _deg_body,
        out_type=jax.ShapeDtypeStruct((NC, NP), jnp.float32),
        mesh=_mesh(),
        scratch_types=[
            pltpu.VMEM((EPT,), jnp.int32),
            pltpu.VMEM((DCH,), jnp.float32),
            pltpu.VMEM((DR, DCH), jnp.int32),
            pltpu.VMEM((RPT,), jnp.float32),
            pltpu.VMEM_SHARED((NP,), jnp.float32),
        ] + [pltpu.SemaphoreType.DMA],
    )
    return f(edge_index)


def _edge_body(ei_hbm, yw_hbm, out_hbm, sidx, didx, rows, *sems):
    isems = sems[:K2]
    gsems = sems[K2:K2 + R]
    ssems = sems[K2 + R:K2 + 2 * R]
    acc_sh = sems[K2 + 2 * R]
    cid = lax.axis_index("c")
    sid = lax.axis_index("s")
    wid = cid * NS + sid
    base = wid * EPT

    def _fire_idx(slot, i):
        pltpu.async_copy(ei_hbm.at[pl.ds(base + i * CH, CH)],
                         sidx.at[slot], isems[slot])
        pltpu.async_copy(ei_hbm.at[pl.ds(N_EDGES + base + i * CH, CH)],
                         didx.at[slot], isems[slot])

    def _wait_idx(slot, i):
        pltpu.make_async_copy(ei_hbm.at[pl.ds(base + i * CH, CH)],
                              sidx.at[slot], isems[slot]).wait()
        pltpu.make_async_copy(ei_hbm.at[pl.ds(N_EDGES + base + i * CH, CH)],
                              didx.at[slot], isems[slot]).wait()

    def _fire_gather(slot, rslot):
        pltpu.async_copy(yw_hbm.at[sidx.at[slot]], rows.at[rslot],
                         gsems[rslot])

    def _wait_gather(slot, rslot):
        pltpu.make_async_copy(yw_hbm.at[sidx.at[slot]], rows.at[rslot],
                              gsems[rslot]).wait()

    def _fire_scat(slot, rslot):
        pltpu.async_copy(rows.at[rslot], acc_sh.at[didx.at[slot]],
                         ssems[rslot], add=True)

    def _wait_scat(slot, rslot):
        pltpu.make_async_copy(rows.at[rslot], acc_sh.at[didx.at[slot]],
                              ssems[rslot]).wait()

    def _fill(i, _):
        r = i // (D // 16)
        c = i % (D // 16)
        rows[0, r, pl.ds(c * 16, 16)] = jnp.zeros((16,), jnp.float32)
        return 0
    lax.fori_loop(0, CH * (D // 16), _fill, 0)
    for k in range(RPT // CH):
        pltpu.sync_copy(rows.at[0], acc_sh.at[pl.ds(sid * RPT + k * CH, CH), :])
    plsc.subcore_barrier()

    for j in range(F):
        _fire_idx(j, j)
    for j in range(A):
        _wait_idx(j, j)
        _fire_gather(j, j)

    def _group(g, _):
        for b in range(K2):
            i = g * K2 + b

            @pl.when(i < NCHUNK)
            def _():
                _wait_gather(b, b % R)
                _fire_scat(b, b % R)

            @pl.when((i >= W) & (i < NCHUNK + W))
            def _():
                _wait_scat((b - W) % K2, (b - W) % R)

            @pl.when(i + A < NCHUNK)
            def _():
                _wait_idx((b + A) % K2, i + A)
                _fire_gather((b + A) % K2, (b + A) % R)

            @pl.when(i + F < NCHUNK)
            def _():
                _fire_idx((b + F) % K2, i + F)
        return 0
    lax.fori_loop(0, NGRP + 1, _group, 0)

    plsc.subcore_barrier()
    pltpu.sync_copy(acc_sh.at[pl.ds(sid * RPT, RPT), :],
                    out_hbm.at[cid, pl.ds(sid * RPT, RPT), :])


def _edge_call(edge_index, yw):
    f = pl.kernel(
        _edge_body,
        out_type=jax.ShapeDtypeStruct((NC, NP, D), jnp.float32),
        mesh=_mesh(),
        scratch_types=(
            [pltpu.VMEM((K2, CH), jnp.int32),
             pltpu.VMEM((K2, CH), jnp.int32),
             pltpu.VMEM((R, CH, D), jnp.float32)]
            + [pltpu.SemaphoreType.DMA] * (K2 + 2 * R)
            + [pltpu.VMEM_SHARED((NP, D), jnp.float32)]
        ),
    )
    return f(edge_index, yw)


def _wgen_body(mw_ref, wih_ref, bih_ref, bhh_ref, wwt_ref, bwt_ref, nw_ref):
    h = MEM
    dn = (((1,), (1,)), ((), ()))
    gi = lax.dot_general(mw_ref[...], wih_ref[...], dn,
                         precision=_HIGH) + bih_ref[...]
    gh = bhh_ref[...]
    r = jax.nn.sigmoid(gi[:, 0:h] + gh[:, 0:h])
    z = jax.nn.sigmoid(gi[:, h:2 * h] + gh[:, h:2 * h])
    n = jnp.tanh(gi[:, 2 * h:] + r * gh[:, 2 * h:])
    um = (1.0 - z) * n
    nw_ref[...] = lax.dot_general(um, wwt_ref[...], dn,
                                  precision=_HIGH) + bwt_ref[...]


def _wgen_call(memory_weights, w_ih, b_ih, b_hh, W_wt, b_wt):
    f = pl.pallas_call(
        _wgen_body,
        out_shape=jax.ShapeDtypeStruct((1, D * D), jnp.float32),
    )
    return f(memory_weights.reshape(1, MEM), w_ih,
             b_ih.reshape(1, 3 * MEM), b_hh.reshape(1, 3 * MEM),
             W_wt, b_wt.reshape(1, D * D))


_NB = 1000


def _deg_sum(deg_ref):
    d = deg_ref[0]
    for c in range(1, NC):
        d = d + deg_ref[c]
    return d


def _yw_body(x_ref, w_ref, deg_ref, yw_ref):
    dis = lax.rsqrt(_deg_sum(deg_ref) + 1.0)
    xw = lax.dot_general(x_ref[...], w_ref[...],
                         (((1,), (1,)), ((), ())), precision=_HIGH)
    yw_ref[...] = xw * dis


def _yw_call(x, W2, deg3):
    grid = N_NODES // _NB
    f = pl.pallas_call(
        _yw_body,
        grid=(grid,),
        in_specs=[
            pl.BlockSpec((_NB, D), lambda j: (j, 0)),
            pl.BlockSpec((D, D), lambda j: (0, 0)),
            pl.BlockSpec((NC, _NB, 1), lambda j: (0, j, 0)),
        ],
        out_specs=pl.BlockSpec((_NB, D), lambda j: (j, 0)),
        out_shape=jax.ShapeDtypeStruct((N_NODES, D), jnp.float32),
    )
    return f(x, W2, deg3)


def _comb_body(parts_ref, yw_ref, deg_ref, bias_ref, out_ref):
    dis = lax.rsqrt(_deg_sum(deg_ref) + 1.0)
    s = parts_ref[0] + yw_ref[...]
    for c in range(1, NC):
        s = s + parts_ref[c]
    out_ref[...] = dis * s + bias_ref[...]


def _comb_call(parts, yw, deg3, gcn_bias):
    grid = N_NODES // _NB
    f = pl.pallas_call(
        _comb_body,
        grid=(grid,),
        in_specs=[
            pl.BlockSpec((NC, _NB, D), lambda j: (0, j, 0)),
            pl.BlockSpec((_NB, D), lambda j: (j, 0)),
            pl.BlockSpec((NC, _NB, 1), lambda j: (0, j, 0)),
            pl.BlockSpec((1, D), lambda j: (0, 0)),
        ],
        out_specs=pl.BlockSpec((_NB, D), lambda j: (j, 0)),
        out_shape=jax.ShapeDtypeStruct((N_NODES, D), jnp.float32),
    )
    return f(parts, yw, deg3, gcn_bias.reshape(1, D))


def kernel(x, edge_index, memory_weights, w_ih, w_hh, b_ih, b_hh,
           W_wt, b_wt, gcn_bias):
    ei_flat = edge_index.reshape(2 * N_EDGES)
    deg = _deg_call(ei_flat)
    deg3 = deg.reshape(NC, NP, 1)
    nw = _wgen_call(memory_weights, w_ih, b_ih, b_hh, W_wt, b_wt)
    W2 = nw.reshape(D, D)
    yw = _yw_call(x, W2, deg3)
    parts = _edge_call(ei_flat, yw)
    return _comb_call(parts, yw, deg3, gcn_bias)


# speedup vs baseline: 1.3303x; 1.0390x over previous
import jax
import jax.numpy as jnp
from jax import lax
from jax.experimental import pallas as pl
from jax.experimental.pallas import tpu as pltpu
from jax.experimental.pallas import tpu_sc as plsc

N_NODES = 10000
N_EDGES = 320000
D = 128
MEM = 256

NC = 1
NS = 16
NW = NC * NS
NP = 10240
RPT = NP // NS
EPT = N_EDGES // NW

CH = 40
NCHUNK = EPT // CH
R = 6
A = 4
W = R - A
F = 8
K2 = 12
NGRP = (NCHUNK + K2 - 1) // K2

DCH = 80
DNCHUNK = EPT // DCH
DR = 8

_HIGH = lax.Precision.HIGHEST


def _mesh():
    return plsc.VectorSubcoreMesh(
        core_axis_name="c", subcore_axis_name="s",
        num_cores=NC, num_subcores=NS)


def _deg_body(ei_hbm, out_hbm, dstf, ones_v, idxr, zb, deg_sh, *sems):
    isem = sems[0]
    cid = lax.axis_index("c")
    sid = lax.axis_index("s")
    wid = cid * NS + sid
    base = wid * EPT

    def _fill(i, _):
        zb[pl.ds(i * 16, 16)] = jnp.zeros((16,), jnp.float32)
        return 0
    lax.fori_loop(0, RPT // 16, _fill, 0)
    for k in range(DCH // 16):
        ones_v[pl.ds(k * 16, 16)] = jnp.ones((16,), jnp.float32)

    pltpu.sync_copy(zb, deg_sh.at[pl.ds(sid * RPT, RPT)])
    plsc.subcore_barrier()

    pltpu.async_copy(ei_hbm.at[pl.ds(N_EDGES + base, EPT)], dstf, isem).wait()

    def _chunk(i, _):
        for k in range(DCH // 16):
            idxr[0, pl.ds(k * 16, 16)] = dstf[pl.ds(i * DCH + k * 16, 16)]
        pltpu.sync_copy(ones_v, deg_sh.at[idxr.at[0]], add=True)
        return 0
    lax.fori_loop(0, DNCHUNK, _chunk, 0)

    plsc.subcore_barrier()
    pltpu.sync_copy(deg_sh.at[pl.ds(sid * RPT, RPT)],
                    out_hbm.at[cid, pl.ds(sid * RPT, RPT)])


def _deg_call(edge_index):
    f = pl.kernel(
        _deg_body,
        out_type=jax.ShapeDtypeStruct((NC, NP), jnp.float32),
        mesh=_mesh(),
        scratch_types=[
            pltpu.VMEM((EPT,), jnp.int32),
            pltpu.VMEM((DCH,), jnp.float32),
            pltpu.VMEM((DR, DCH), jnp.int32),
            pltpu.VMEM((RPT,), jnp.float32),
            pltpu.VMEM_SHARED((NP,), jnp.float32),
        ] + [pltpu.SemaphoreType.DMA],
    )
    return f(edge_index)


def _edge_body(ei_hbm, yw_hbm, out_hbm, sidx, didx, rows, *sems):
    isems = sems[:K2]
    gsems = sems[K2:K2 + R]
    ssems = sems[K2 + R:K2 + 2 * R]
    acc_sh = sems[K2 + 2 * R]
    cid = lax.axis_index("c")
    sid = lax.axis_index("s")
    wid = cid * NS + sid
    base = wid * EPT

    def _fire_idx(slot, i):
        pltpu.async_copy(ei_hbm.at[pl.ds(base + i * CH, CH)],
                         sidx.at[slot], isems[slot])
        pltpu.async_copy(ei_hbm.at[pl.ds(N_EDGES + base + i * CH, CH)],
                         didx.at[slot], isems[slot])

    def _wait_idx(slot, i):
        pltpu.make_async_copy(ei_hbm.at[pl.ds(base + i * CH, CH)],
                              sidx.at[slot], isems[slot]).wait()
        pltpu.make_async_copy(ei_hbm.at[pl.ds(N_EDGES + base + i * CH, CH)],
                              didx.at[slot], isems[slot]).wait()

    def _fire_gather(slot, rslot):
        pltpu.async_copy(yw_hbm.at[sidx.at[slot]], rows.at[rslot],
                         gsems[rslot])

    def _wait_gather(slot, rslot):
        pltpu.make_async_copy(yw_hbm.at[sidx.at[slot]], rows.at[rslot],
                              gsems[rslot]).wait()

    def _fire_scat(slot, rslot):
        pltpu.async_copy(rows.at[rslot], acc_sh.at[didx.at[slot]],
                         ssems[rslot], add=True)

    def _wait_scat(slot, rslot):
        pltpu.make_async_copy(rows.at[rslot], acc_sh.at[didx.at[slot]],
                              ssems[rslot]).wait()

    def _fill(i, _):
        r = i // (D // 16)
        c = i % (D // 16)
        rows[0, r, pl.ds(c * 16, 16)] = jnp.zeros((16,), jnp.float32)
        return 0
    lax.fori_loop(0, CH * (D // 16), _fill, 0)
    for k in range(RPT // CH):
        pltpu.sync_copy(rows.at[0], acc_sh.at[pl.ds(sid * RPT + k * CH, CH), :])
    plsc.subcore_barrier()

    for j in range(F):
        _fire_idx(j, j)
    for j in range(A):
        _wait_idx(j, j)
        _fire_gather(j, j)

    def _group(g, _):
        for b in range(K2):
            i = g * K2 + b

            @pl.when(i < NCHUNK)
            def _():
                _wait_gather(b, b % R)
                _fire_scat(b, b % R)

            @pl.when((i >= W) & (i < NCHUNK + W))
            def _():
                _wait_scat((b - W) % K2, (b - W) % R)

            @pl.when(i + A < NCHUNK)
            def _():
                _wait_idx((b + A) % K2, i + A)
                _fire_gather((b + A) % K2, (b + A) % R)

            @pl.when(i + F < NCHUNK)
            def _():
                _fire_idx((b + F) % K2, i + F)
        return 0
    lax.fori_loop(0, NGRP + 1, _group, 0)

    plsc.subcore_barrier()
    pltpu.sync_copy(acc_sh.at[pl.ds(sid * RPT, RPT), :],
                    out_hbm.at[cid, pl.ds(sid * RPT, RPT), :])


def _edge_call(edge_index, yw):
    f = pl.kernel(
        _edge_body,
        out_type=jax.ShapeDtypeStruct((NC, NP, D), jnp.float32),
        mesh=_mesh(),
        scratch_types=(
            [pltpu.VMEM((K2, CH), jnp.int32),
             pltpu.VMEM((K2, CH), jnp.int32),
             pltpu.VMEM((R, CH, D), jnp.float32)]
            + [pltpu.SemaphoreType.DMA] * (K2 + 2 * R)
            + [pltpu.VMEM_SHARED((NP, D), jnp.float32)]
        ),
    )
    return f(edge_index, yw)


def _wgen_body(mw_ref, wih_ref, bih_ref, bhh_ref, wwt_ref, bwt_ref, nw_ref):
    h = MEM
    dn = (((1,), (1,)), ((), ()))
    gi = lax.dot_general(mw_ref[...], wih_ref[...], dn,
                         precision=_HIGH) + bih_ref[...]
    gh = bhh_ref[...]
    r = jax.nn.sigmoid(gi[:, 0:h] + gh[:, 0:h])
    z = jax.nn.sigmoid(gi[:, h:2 * h] + gh[:, h:2 * h])
    n = jnp.tanh(gi[:, 2 * h:] + r * gh[:, 2 * h:])
    um = (1.0 - z) * n
    nw_ref[...] = lax.dot_general(um, wwt_ref[...], dn,
                                  precision=_HIGH) + bwt_ref[...]


def _wgen_call(memory_weights, w_ih, b_ih, b_hh, W_wt, b_wt):
    f = pl.pallas_call(
        _wgen_body,
        out_shape=jax.ShapeDtypeStruct((1, D * D), jnp.float32),
    )
    return f(memory_weights.reshape(1, MEM), w_ih,
             b_ih.reshape(1, 3 * MEM), b_hh.reshape(1, 3 * MEM),
             W_wt, b_wt.reshape(1, D * D))


_NB = 1000


def _deg_sum(deg_ref):
    d = deg_ref[0]
    for c in range(1, NC):
        d = d + deg_ref[c]
    return d


def _yw_body(x_ref, w_ref, deg_ref, yw_ref):
    dis = lax.rsqrt(_deg_sum(deg_ref) + 1.0)
    xw = lax.dot_general(x_ref[...], w_ref[...],
                         (((1,), (1,)), ((), ())), precision=_HIGH)
    yw_ref[...] = xw * dis


def _yw_call(x, W2, deg3):
    grid = N_NODES // _NB
    f = pl.pallas_call(
        _yw_body,
        grid=(grid,),
        in_specs=[
            pl.BlockSpec((_NB, D), lambda j: (j, 0)),
            pl.BlockSpec((D, D), lambda j: (0, 0)),
            pl.BlockSpec((NC, _NB, 1), lambda j: (0, j, 0)),
        ],
        out_specs=pl.BlockSpec((_NB, D), lambda j: (j, 0)),
        out_shape=jax.ShapeDtypeStruct((N_NODES, D), jnp.float32),
    )
    return f(x, W2, deg3)


def _comb_body(parts_ref, yw_ref, deg_ref, bias_ref, out_ref):
    dis = lax.rsqrt(_deg_sum(deg_ref) + 1.0)
    s = parts_ref[0] + yw_ref[...]
    for c in range(1, NC):
        s = s + parts_ref[c]
    out_ref[...] = dis * s + bias_ref[...]


def _comb_call(parts, yw, deg3, gcn_bias):
    grid = N_NODES // _NB
    f = pl.pallas_call(
        _comb_body,
        grid=(grid,),
        in_specs=[
            pl.BlockSpec((NC, _NB, D), lambda j: (0, j, 0)),
            pl.BlockSpec((_NB, D), lambda j: (j, 0)),
            pl.BlockSpec((NC, _NB, 1), lambda j: (0, j, 0)),
            pl.BlockSpec((1, D), lambda j: (0, 0)),
        ],
        out_specs=pl.BlockSpec((_NB, D), lambda j: (j, 0)),
        out_shape=jax.ShapeDtypeStruct((N_NODES, D), jnp.float32),
    )
    return f(parts, yw, deg3, gcn_bias.reshape(1, D))


def kernel(x, edge_index, memory_weights, w_ih, w_hh, b_ih, b_hh,
           W_wt, b_wt, gcn_bias):
    ei_flat = edge_index.reshape(2 * N_EDGES)
    deg = _deg_call(ei_flat)
    deg3 = deg.reshape(NC, NP, 1)
    nw = _wgen_call(memory_weights, w_ih, b_ih, b_hh, W_wt, b_wt)
    W2 = nw.reshape(D, D)
    yw = _yw_call(x, W2, deg3)
    parts = _edge_call(ei_flat, yw)
    return _comb_call(parts, yw, deg3, gcn_bias)


# speedup vs baseline: 1.3679x; 1.0283x over previous
import jax
import jax.numpy as jnp
from jax import lax
from jax.experimental import pallas as pl
from jax.experimental.pallas import tpu as pltpu
from jax.experimental.pallas import tpu_sc as plsc

N_NODES = 10000
N_EDGES = 320000
D = 128
MEM = 256

NC = 1
NS = 16
NW = NC * NS
NP = 10240
RPT = NP // NS
EPT = N_EDGES // NW

CH = 40
NCHUNK = EPT // CH
R = 6
A = 5
W = R - A
F = 8
K2 = 12
NGRP = (NCHUNK + K2 - 1) // K2

DCH = 80
DNCHUNK = EPT // DCH
DR = 8

_HIGH = lax.Precision.HIGHEST


def _mesh():
    return plsc.VectorSubcoreMesh(
        core_axis_name="c", subcore_axis_name="s",
        num_cores=NC, num_subcores=NS)


def _deg_body(ei_hbm, out_hbm, dstf, ones_v, idxr, zb, deg_sh, *sems):
    isem = sems[0]
    cid = lax.axis_index("c")
    sid = lax.axis_index("s")
    wid = cid * NS + sid
    base = wid * EPT

    def _fill(i, _):
        zb[pl.ds(i * 16, 16)] = jnp.zeros((16,), jnp.float32)
        return 0
    lax.fori_loop(0, RPT // 16, _fill, 0)
    for k in range(DCH // 16):
        ones_v[pl.ds(k * 16, 16)] = jnp.ones((16,), jnp.float32)

    pltpu.sync_copy(zb, deg_sh.at[pl.ds(sid * RPT, RPT)])
    plsc.subcore_barrier()

    pltpu.async_copy(ei_hbm.at[pl.ds(N_EDGES + base, EPT)], dstf, isem).wait()

    def _chunk(i, _):
        for k in range(DCH // 16):
            idxr[0, pl.ds(k * 16, 16)] = dstf[pl.ds(i * DCH + k * 16, 16)]
        pltpu.sync_copy(ones_v, deg_sh.at[idxr.at[0]], add=True)
        return 0
    lax.fori_loop(0, DNCHUNK, _chunk, 0)

    plsc.subcore_barrier()
    pltpu.sync_copy(deg_sh.at[pl.ds(sid * RPT, RPT)],
                    out_hbm.at[cid, pl.ds(sid * RPT, RPT)])


def _deg_call(edge_index):
    f = pl.kernel(
        _deg_body,
        out_type=jax.ShapeDtypeStruct((NC, NP), jnp.float32),
        mesh=_mesh(),
        scratch_types=[
            pltpu.VMEM((EPT,), jnp.int32),
            pltpu.VMEM((DCH,), jnp.float32),
            pltpu.VMEM((DR, DCH), jnp.int32),
            pltpu.VMEM((RPT,), jnp.float32),
            pltpu.VMEM_SHARED((NP,), jnp.float32),
        ] + [pltpu.SemaphoreType.DMA],
    )
    return f(edge_index)


def _edge_body(ei_hbm, yw_hbm, out_hbm, sidx, didx, rows, *sems):
    isems = sems[:K2]
    gsems = sems[K2:K2 + R]
    ssems = sems[K2 + R:K2 + 2 * R]
    acc_sh = sems[K2 + 2 * R]
    cid = lax.axis_index("c")
    sid = lax.axis_index("s")
    wid = cid * NS + sid
    base = wid * EPT

    def _fire_idx(slot, i):
        pltpu.async_copy(ei_hbm.at[pl.ds(base + i * CH, CH)],
                         sidx.at[slot], isems[slot])
        pltpu.async_copy(ei_hbm.at[pl.ds(N_EDGES + base + i * CH, CH)],
                         didx.at[slot], isems[slot])

    def _wait_idx(slot, i):
        pltpu.make_async_copy(ei_hbm.at[pl.ds(base + i * CH, CH)],
                              sidx.at[slot], isems[slot]).wait()
        pltpu.make_async_copy(ei_hbm.at[pl.ds(N_EDGES + base + i * CH, CH)],
                              didx.at[slot], isems[slot]).wait()

    def _fire_gather(slot, rslot):
        pltpu.async_copy(yw_hbm.at[sidx.at[slot]], rows.at[rslot],
                         gsems[rslot])

    def _wait_gather(slot, rslot):
        pltpu.make_async_copy(yw_hbm.at[sidx.at[slot]], rows.at[rslot],
                              gsems[rslot]).wait()

    def _fire_scat(slot, rslot):
        pltpu.async_copy(rows.at[rslot], acc_sh.at[didx.at[slot]],
                         ssems[rslot], add=True)

    def _wait_scat(slot, rslot):
        pltpu.make_async_copy(rows.at[rslot], acc_sh.at[didx.at[slot]],
                              ssems[rslot]).wait()

    def _fill(i, _):
        r = i // (D // 16)
        c = i % (D // 16)
        rows[0, r, pl.ds(c * 16, 16)] = jnp.zeros((16,), jnp.float32)
        return 0
    lax.fori_loop(0, CH * (D // 16), _fill, 0)
    for k in range(RPT // CH):
        pltpu.sync_copy(rows.at[0], acc_sh.at[pl.ds(sid * RPT + k * CH, CH), :])
    plsc.subcore_barrier()

    for j in range(F):
        _fire_idx(j, j)
    for j in range(A):
        _wait_idx(j, j)
        _fire_gather(j, j)

    def _group(g, _):
        for b in range(K2):
            i = g * K2 + b

            @pl.when(i < NCHUNK)
            def _():
                _wait_gather(b, b % R)
                _fire_scat(b, b % R)

            @pl.when((i >= W) & (i < NCHUNK + W))
            def _():
                _wait_scat((b - W) % K2, (b - W) % R)

            @pl.when(i + A < NCHUNK)
            def _():
                _wait_idx((b + A) % K2, i + A)
                _fire_gather((b + A) % K2, (b + A) % R)

            @pl.when(i + F < NCHUNK)
            def _():
                _fire_idx((b + F) % K2, i + F)
        return 0
    lax.fori_loop(0, NGRP + 1, _group, 0)

    plsc.subcore_barrier()
    pltpu.sync_copy(acc_sh.at[pl.ds(sid * RPT, RPT), :],
                    out_hbm.at[cid, pl.ds(sid * RPT, RPT), :])


def _edge_call(edge_index, yw):
    f = pl.kernel(
        _edge_body,
        out_type=jax.ShapeDtypeStruct((NC, NP, D), jnp.float32),
        mesh=_mesh(),
        scratch_types=(
            [pltpu.VMEM((K2, CH), jnp.int32),
             pltpu.VMEM((K2, CH), jnp.int32),
             pltpu.VMEM((R, CH, D), jnp.float32)]
            + [pltpu.SemaphoreType.DMA] * (K2 + 2 * R)
            + [pltpu.VMEM_SHARED((NP, D), jnp.float32)]
        ),
    )
    return f(edge_index, yw)


def _wgen_body(mw_ref, wih_ref, bih_ref, bhh_ref, wwt_ref, bwt_ref, nw_ref):
    h = MEM
    dn = (((1,), (1,)), ((), ()))
    gi = lax.dot_general(mw_ref[...], wih_ref[...], dn,
                         precision=_HIGH) + bih_ref[...]
    gh = bhh_ref[...]
    r = jax.nn.sigmoid(gi[:, 0:h] + gh[:, 0:h])
    z = jax.nn.sigmoid(gi[:, h:2 * h] + gh[:, h:2 * h])
    n = jnp.tanh(gi[:, 2 * h:] + r * gh[:, 2 * h:])
    um = (1.0 - z) * n
    nw_ref[...] = lax.dot_general(um, wwt_ref[...], dn,
                                  precision=_HIGH) + bwt_ref[...]


def _wgen_call(memory_weights, w_ih, b_ih, b_hh, W_wt, b_wt):
    f = pl.pallas_call(
        _wgen_body,
        out_shape=jax.ShapeDtypeStruct((1, D * D), jnp.float32),
    )
    return f(memory_weights.reshape(1, MEM), w_ih,
             b_ih.reshape(1, 3 * MEM), b_hh.reshape(1, 3 * MEM),
             W_wt, b_wt.reshape(1, D * D))


_NB = 1000


def _deg_sum(deg_ref):
    d = deg_ref[0]
    for c in range(1, NC):
        d = d + deg_ref[c]
    return d


def _yw_body(x_ref, w_ref, deg_ref, yw_ref):
    dis = lax.rsqrt(_deg_sum(deg_ref) + 1.0)
    xw = lax.dot_general(x_ref[...], w_ref[...],
                         (((1,), (1,)), ((), ())), precision=_HIGH)
    yw_ref[...] = xw * dis


def _yw_call(x, W2, deg3):
    grid = N_NODES // _NB
    f = pl.pallas_call(
        _yw_body,
        grid=(grid,),
        in_specs=[
            pl.BlockSpec((_NB, D), lambda j: (j, 0)),
            pl.BlockSpec((D, D), lambda j: (0, 0)),
            pl.BlockSpec((NC, _NB, 1), lambda j: (0, j, 0)),
        ],
        out_specs=pl.BlockSpec((_NB, D), lambda j: (j, 0)),
        out_shape=jax.ShapeDtypeStruct((N_NODES, D), jnp.float32),
    )
    return f(x, W2, deg3)


def _comb_body(parts_ref, yw_ref, deg_ref, bias_ref, out_ref):
    dis = lax.rsqrt(_deg_sum(deg_ref) + 1.0)
    s = parts_ref[0] + yw_ref[...]
    for c in range(1, NC):
        s = s + parts_ref[c]
    out_ref[...] = dis * s + bias_ref[...]


def _comb_call(parts, yw, deg3, gcn_bias):
    grid = N_NODES // _NB
    f = pl.pallas_call(
        _comb_body,
        grid=(grid,),
        in_specs=[
            pl.BlockSpec((NC, _NB, D), lambda j: (0, j, 0)),
            pl.BlockSpec((_NB, D), lambda j: (j, 0)),
            pl.BlockSpec((NC, _NB, 1), lambda j: (0, j, 0)),
            pl.BlockSpec((1, D), lambda j: (0, 0)),
        ],
        out_specs=pl.BlockSpec((_NB, D), lambda j: (j, 0)),
        out_shape=jax.ShapeDtypeStruct((N_NODES, D), jnp.float32),
    )
    return f(parts, yw, deg3, gcn_bias.reshape(1, D))


def kernel(x, edge_index, memory_weights, w_ih, w_hh, b_ih, b_hh,
           W_wt, b_wt, gcn_bias):
    ei_flat = edge_index.reshape(2 * N_EDGES)
    deg = _deg_call(ei_flat)
    deg3 = deg.reshape(NC, NP, 1)
    nw = _wgen_call(memory_weights, w_ih, b_ih, b_hh, W_wt, b_wt)
    W2 = nw.reshape(D, D)
    yw = _yw_call(x, W2, deg3)
    parts = _edge_call(ei_flat, yw)
    return _comb_call(parts, yw, deg3, gcn_bias)


# speedup vs baseline: 1.4166x; 1.0356x over previous
import jax
import jax.numpy as jnp
from jax import lax
from jax.experimental import pallas as pl
from jax.experimental.pallas import tpu as pltpu
from jax.experimental.pallas import tpu_sc as plsc

N_NODES = 10000
N_EDGES = 320000
D = 128
MEM = 256

NC = 1
NS = 16
NW = NC * NS
NP = 10240
RPT = NP // NS
EPT = N_EDGES // NW

CH = 40
NCHUNK = EPT // CH
R = 6
A = 5
W = R - A
F = 8
K2 = 12
NGRP = (NCHUNK + K2 - 1) // K2

DCH = 80
DNCHUNK = EPT // DCH
DR = 4

_HIGH = lax.Precision.HIGHEST


def _mesh():
    return plsc.VectorSubcoreMesh(
        core_axis_name="c", subcore_axis_name="s",
        num_cores=NC, num_subcores=NS)


def _deg_body(ei_hbm, out_hbm, dstf, ones_v, idxr, zb, deg_sh, *sems):
    isem = sems[0]
    ssems = sems[1:1 + DR]
    cid = lax.axis_index("c")
    sid = lax.axis_index("s")
    wid = cid * NS + sid
    base = wid * EPT

    def _fill(i, _):
        zb[pl.ds(i * 16, 16)] = jnp.zeros((16,), jnp.float32)
        return 0
    lax.fori_loop(0, RPT // 16, _fill, 0)
    for k in range(DCH // 16):
        ones_v[pl.ds(k * 16, 16)] = jnp.ones((16,), jnp.float32)

    pltpu.sync_copy(zb, deg_sh.at[pl.ds(sid * RPT, RPT)])
    plsc.subcore_barrier()

    pltpu.async_copy(ei_hbm.at[pl.ds(N_EDGES + base, EPT)], dstf, isem).wait()

    def _group(g, _):
        descs = []
        for b in range(DR):
            i = g * DR + b
            for k in range(DCH // 16):
                idxr[b, pl.ds(k * 16, 16)] = \
                    dstf[pl.ds(i * DCH + k * 16, 16)]
            descs.append(pltpu.async_copy(ones_v, deg_sh.at[idxr.at[b]],
                                          ssems[b], add=True))
        for dsc in descs:
            dsc.wait()
        return 0
    lax.fori_loop(0, DNCHUNK // DR, _group, 0)

    def _tail(i, _):
        for k in range(DCH // 16):
            idxr[0, pl.ds(k * 16, 16)] = dstf[pl.ds(i * DCH + k * 16, 16)]
        pltpu.sync_copy(ones_v, deg_sh.at[idxr.at[0]], add=True)
        return 0
    lax.fori_loop((DNCHUNK // DR) * DR, DNCHUNK, _tail, 0)

    plsc.subcore_barrier()
    pltpu.sync_copy(deg_sh.at[pl.ds(sid * RPT, RPT)],
                    out_hbm.at[cid, pl.ds(sid * RPT, RPT)])


def _deg_call(edge_index):
    f = pl.kernel(
        _deg_body,
        out_type=jax.ShapeDtypeStruct((NC, NP), jnp.float32),
        mesh=_mesh(),
        scratch_types=[
            pltpu.VMEM((EPT,), jnp.int32),
            pltpu.VMEM((DCH,), jnp.float32),
            pltpu.VMEM((DR, DCH), jnp.int32),
            pltpu.VMEM((RPT,), jnp.float32),
            pltpu.VMEM_SHARED((NP,), jnp.float32),
        ] + [pltpu.SemaphoreType.DMA] * (1 + DR),
    )
    return f(edge_index)


def _edge_body(ei_hbm, yw_hbm, out_hbm, sidx, didx, rows, *sems):
    isems = sems[:K2]
    gsems = sems[K2:K2 + R]
    ssems = sems[K2 + R:K2 + 2 * R]
    acc_sh = sems[K2 + 2 * R]
    cid = lax.axis_index("c")
    sid = lax.axis_index("s")
    wid = cid * NS + sid
    base = wid * EPT

    def _fire_idx(slot, i):
        pltpu.async_copy(ei_hbm.at[pl.ds(base + i * CH, CH)],
                         sidx.at[slot], isems[slot])
        pltpu.async_copy(ei_hbm.at[pl.ds(N_EDGES + base + i * CH, CH)],
                         didx.at[slot], isems[slot])

    def _wait_idx(slot, i):
        pltpu.make_async_copy(ei_hbm.at[pl.ds(base + i * CH, CH)],
                              sidx.at[slot], isems[slot]).wait()
        pltpu.make_async_copy(ei_hbm.at[pl.ds(N_EDGES + base + i * CH, CH)],
                              didx.at[slot], isems[slot]).wait()

    def _fire_gather(slot, rslot):
        pltpu.async_copy(yw_hbm.at[sidx.at[slot]], rows.at[rslot],
                         gsems[rslot])

    def _wait_gather(slot, rslot):
        pltpu.make_async_copy(yw_hbm.at[sidx.at[slot]], rows.at[rslot],
                              gsems[rslot]).wait()

    def _fire_scat(slot, rslot):
        pltpu.async_copy(rows.at[rslot], acc_sh.at[didx.at[slot]],
                         ssems[rslot], add=True)

    def _wait_scat(slot, rslot):
        pltpu.make_async_copy(rows.at[rslot], acc_sh.at[didx.at[slot]],
                              ssems[rslot]).wait()

    def _fill(i, _):
        r = i // (D // 16)
        c = i % (D // 16)
        rows[0, r, pl.ds(c * 16, 16)] = jnp.zeros((16,), jnp.float32)
        return 0
    lax.fori_loop(0, CH * (D // 16), _fill, 0)
    for k in range(RPT // CH):
        pltpu.sync_copy(rows.at[0], acc_sh.at[pl.ds(sid * RPT + k * CH, CH), :])
    plsc.subcore_barrier()

    for j in range(F):
        _fire_idx(j, j)
    for j in range(A):
        _wait_idx(j, j)
        _fire_gather(j, j)

    def _group(g, _):
        for b in range(K2):
            i = g * K2 + b

            @pl.when(i < NCHUNK)
            def _():
                _wait_gather(b, b % R)
                _fire_scat(b, b % R)

            @pl.when((i >= W) & (i < NCHUNK + W))
            def _():
                _wait_scat((b - W) % K2, (b - W) % R)

            @pl.when(i + A < NCHUNK)
            def _():
                _wait_idx((b + A) % K2, i + A)
                _fire_gather((b + A) % K2, (b + A) % R)

            @pl.when(i + F < NCHUNK)
            def _():
                _fire_idx((b + F) % K2, i + F)
        return 0
    lax.fori_loop(0, NGRP + 1, _group, 0)

    plsc.subcore_barrier()
    pltpu.sync_copy(acc_sh.at[pl.ds(sid * RPT, RPT), :],
                    out_hbm.at[cid, pl.ds(sid * RPT, RPT), :])


def _edge_call(edge_index, yw):
    f = pl.kernel(
        _edge_body,
        out_type=jax.ShapeDtypeStruct((NC, NP, D), jnp.float32),
        mesh=_mesh(),
        scratch_types=(
            [pltpu.VMEM((K2, CH), jnp.int32),
             pltpu.VMEM((K2, CH), jnp.int32),
             pltpu.VMEM((R, CH, D), jnp.float32)]
            + [pltpu.SemaphoreType.DMA] * (K2 + 2 * R)
            + [pltpu.VMEM_SHARED((NP, D), jnp.float32)]
        ),
    )
    return f(edge_index, yw)


def _wgen_body(mw_ref, wih_ref, bih_ref, bhh_ref, wwt_ref, bwt_ref, nw_ref):
    h = MEM
    dn = (((1,), (1,)), ((), ()))
    gi = lax.dot_general(mw_ref[...], wih_ref[...], dn,
                         precision=_HIGH) + bih_ref[...]
    gh = bhh_ref[...]
    r = jax.nn.sigmoid(gi[:, 0:h] + gh[:, 0:h])
    z = jax.nn.sigmoid(gi[:, h:2 * h] + gh[:, h:2 * h])
    n = jnp.tanh(gi[:, 2 * h:] + r * gh[:, 2 * h:])
    um = (1.0 - z) * n
    nw_ref[...] = lax.dot_general(um, wwt_ref[...], dn,
                                  precision=_HIGH) + bwt_ref[...]


def _wgen_call(memory_weights, w_ih, b_ih, b_hh, W_wt, b_wt):
    f = pl.pallas_call(
        _wgen_body,
        out_shape=jax.ShapeDtypeStruct((1, D * D), jnp.float32),
    )
    return f(memory_weights.reshape(1, MEM), w_ih,
             b_ih.reshape(1, 3 * MEM), b_hh.reshape(1, 3 * MEM),
             W_wt, b_wt.reshape(1, D * D))


_NB = 1000


def _deg_sum(deg_ref):
    d = deg_ref[0]
    for c in range(1, NC):
        d = d + deg_ref[c]
    return d


def _yw_body(x_ref, w_ref, deg_ref, yw_ref):
    dis = lax.rsqrt(_deg_sum(deg_ref) + 1.0)
    xw = lax.dot_general(x_ref[...], w_ref[...],
                         (((1,), (1,)), ((), ())), precision=_HIGH)
    yw_ref[...] = xw * dis


def _yw_call(x, W2, deg3):
    grid = N_NODES // _NB
    f = pl.pallas_call(
        _yw_body,
        grid=(grid,),
        in_specs=[
            pl.BlockSpec((_NB, D), lambda j: (j, 0)),
            pl.BlockSpec((D, D), lambda j: (0, 0)),
            pl.BlockSpec((NC, _NB, 1), lambda j: (0, j, 0)),
        ],
        out_specs=pl.BlockSpec((_NB, D), lambda j: (j, 0)),
        out_shape=jax.ShapeDtypeStruct((N_NODES, D), jnp.float32),
    )
    return f(x, W2, deg3)


def _comb_body(parts_ref, yw_ref, deg_ref, bias_ref, out_ref):
    dis = lax.rsqrt(_deg_sum(deg_ref) + 1.0)
    s = parts_ref[0] + yw_ref[...]
    for c in range(1, NC):
        s = s + parts_ref[c]
    out_ref[...] = dis * s + bias_ref[...]


def _comb_call(parts, yw, deg3, gcn_bias):
    grid = N_NODES // _NB
    f = pl.pallas_call(
        _comb_body,
        grid=(grid,),
        in_specs=[
            pl.BlockSpec((NC, _NB, D), lambda j: (0, j, 0)),
            pl.BlockSpec((_NB, D), lambda j: (j, 0)),
            pl.BlockSpec((NC, _NB, 1), lambda j: (0, j, 0)),
            pl.BlockSpec((1, D), lambda j: (0, 0)),
        ],
        out_specs=pl.BlockSpec((_NB, D), lambda j: (j, 0)),
        out_shape=jax.ShapeDtypeStruct((N_NODES, D), jnp.float32),
    )
    return f(parts, yw, deg3, gcn_bias.reshape(1, D))


def kernel(x, edge_index, memory_weights, w_ih, w_hh, b_ih, b_hh,
           W_wt, b_wt, gcn_bias):
    ei_flat = edge_index.reshape(2 * N_EDGES)
    deg = _deg_call(ei_flat)
    deg3 = deg.reshape(NC, NP, 1)
    nw = _wgen_call(memory_weights, w_ih, b_ih, b_hh, W_wt, b_wt)
    W2 = nw.reshape(D, D)
    yw = _yw_call(x, W2, deg3)
    parts = _edge_call(ei_flat, yw)
    return _comb_call(parts, yw, deg3, gcn_bias)
